# baseline (speedup 1.0000x reference)
import functools

import jax
import jax.numpy as jnp
from jax import lax
from jax.experimental import pallas as pl
from jax.experimental.pallas import tpu as pltpu
from jax.experimental.pallas import tpu_sc as plsc

B = 16
S = 4096
C = 1024
L = 16
NS = 16
NC = 2
NW = NC * NS
R = 16
NBUF = 4
CB = C // L


def _sc_body(value_h, sent_h, out_h, buf, acc, sentv, tmp, obuf, shared, sem):
    c = lax.axis_index("c")
    s = lax.axis_index("s")
    wid = c * NS + s

    zero16 = jnp.zeros((L,), jnp.float32)

    def zacc(i, carry):
        j = i // CB
        k = i - j * CB
        acc[j, pl.ds(k * L, L)] = zero16
        return carry

    lax.fori_loop(0, B * CB, zacc, 0)

    pltpu.sync_copy(sent_h, sentv)
    lens = sentv[...]
    csum = plsc.cumsum(lens)
    off = csum - lens
    total = jnp.sum(lens)
    chunk = (total + NW - 1) // NW
    start = wid * chunk
    end = jnp.minimum(start + chunk, total)

    iota = lax.iota(jnp.int32, L)

    def ext(v, j):
        return jnp.sum(jnp.where(iota == j, v, 0))

    def dma_start(j, r0, q):
        pltpu.async_copy(value_h.at[j, pl.ds(r0, R), :], buf.at[q], sem.at[q])

    def dma_wait(j, q):
        pltpu.make_async_copy(value_h.at[j, pl.ds(0, R), :],
                              buf.at[q], sem.at[q]).wait()

    def batch_body(j, carry):
        oj = ext(off, j)
        lj = ext(lens, j)
        lo = jnp.maximum(start - oj, 0)
        hi = jnp.minimum(end - oj, lj)
        lo8 = (lo // 8) * 8
        n = jnp.where(hi > lo, hi - lo8, 0)
        nch = (n + R - 1) // R

        for q in range(NBUF - 1):
            @pl.when(q < nch)
            def _(q=q):
                dma_start(j, lo8 + q * R, q)

        def chunk_body(i, carry2):
            p = i % NBUF

            @pl.when(i + (NBUF - 1) < nch)
            def _():
                dma_start(j, lo8 + (i + NBUF - 1) * R, (i + NBUF - 1) % NBUF)

            dma_wait(j, p)
            r0 = lo8 + i * R
            a = jnp.maximum(lo - r0, 0)
            b2 = jnp.minimum(hi - r0, R)

            def row_body(k, carry3):
                for cc in range(CB):
                    plsc.addupdate(acc.at[j, pl.ds(cc * L, L)],
                                   buf[p, k, pl.ds(cc * L, L)])
                return carry3

            lax.fori_loop(a, b2, row_body, 0)
            return carry2

        lax.fori_loop(0, nch, chunk_body, 0)
        return carry

    lax.fori_loop(0, B, batch_body, 0)

    pltpu.sync_copy(acc, shared.at[s])
    plsc.subcore_barrier()

    pltpu.sync_copy(shared.at[0, s], obuf)

    def red_body(w, carry):
        pltpu.sync_copy(shared.at[w, s], tmp)

        def add_body(i, carry2):
            obuf[pl.ds(i * L, L)] = obuf[pl.ds(i * L, L)] + tmp[pl.ds(i * L, L)]
            return carry2

        lax.fori_loop(0, CB, add_body, 0)
        return carry

    lax.fori_loop(1, NS, red_body, 0)
    pltpu.sync_copy(obuf, out_h.at[c, s, pl.ds(0, C)])


_sc_partial = functools.partial(
    pl.kernel,
    out_type=jax.ShapeDtypeStruct((NC, B, C), jnp.float32),
    mesh=plsc.VectorSubcoreMesh(core_axis_name="c", subcore_axis_name="s"),
    compiler_params=pltpu.CompilerParams(needs_layout_passes=False),
    scratch_types=[
        pltpu.VMEM((NBUF, R, C), jnp.float32),
        pltpu.VMEM((B, C), jnp.float32),
        pltpu.VMEM((L,), jnp.int32),
        pltpu.VMEM((C,), jnp.float32),
        pltpu.VMEM((C,), jnp.float32),
        pltpu.VMEM_SHARED((NS, B, C), jnp.float32),
        pltpu.SemaphoreType.DMA((NBUF,)),
    ],
)(_sc_body)


def _comb_body(a_ref, l_ref, o_ref):
    tot = a_ref[0] + a_ref[1]
    o_ref[...] = (tot / l_ref[...]).reshape(B, 1, C)


_combine = pl.pallas_call(
    _comb_body,
    out_shape=jax.ShapeDtypeStruct((B, 1, C), jnp.float32),
)


def kernel(value, sent_len):
    sc_part = _sc_partial(value, sent_len)
    lenf = sent_len.astype(jnp.float32).reshape(B, 1)
    return _combine(sc_part, lenf)


# speedup vs baseline: 3.2106x; 3.2106x over previous
import functools

import jax
import jax.numpy as jnp
from jax import lax
from jax.experimental import pallas as pl
from jax.experimental.pallas import tpu as pltpu
from jax.experimental.pallas import tpu_sc as plsc

B = 16
S = 4096
C = 1024
L = 16
NS = 16
HALF = C // 2
R = 32
NBUF = 4
CB = HALF // L


def _sc_body(value_h, sent_h, out_h, buf, acc, sentv, invv, tmp, obuf,
             shared, sem):
    c = lax.axis_index("c")
    s = lax.axis_index("s")
    ch0 = c * HALF

    pltpu.sync_copy(sent_h, sentv)
    lens = sentv[...]
    invv[...] = 1.0 / lens.astype(jnp.float32)
    csum = plsc.cumsum(lens)
    off = csum - lens
    total = jnp.sum(lens)
    chunk = (total + NS - 1) // NS
    start = s * chunk
    end = jnp.minimum(start + chunk, total)

    iota = lax.iota(jnp.int32, L)

    def ext(v, j):
        return jnp.sum(jnp.where(iota == j, v, 0))

    def dma_start(j, r0, q):
        pltpu.async_copy(value_h.at[j, pl.ds(r0, R), pl.ds(ch0, HALF)],
                         buf.at[q], sem.at[q])

    def dma_wait(j, q):
        pltpu.make_async_copy(value_h.at[j, pl.ds(0, R), pl.ds(ch0, HALF)],
                              buf.at[q], sem.at[q]).wait()

    zeros16 = jnp.zeros((L,), jnp.float32)

    def batch_body(j, carry):
        oj = ext(off, j)
        lj = ext(lens, j)
        lo = jnp.maximum(start - oj, 0)
        hi = jnp.minimum(end - oj, lj)
        lo8 = (lo // 8) * 8
        n = jnp.where(hi > lo, hi - lo8, 0)
        nch = (n + R - 1) // R

        for q in range(NBUF - 1):
            @pl.when(q < nch)
            def _(q=q):
                dma_start(j, lo8 + q * R, q)

        def chunk_body(i, vecs):
            p = i % NBUF

            @pl.when(i + (NBUF - 1) < nch)
            def _():
                dma_start(j, lo8 + (i + NBUF - 1) * R, (i + NBUF - 1) % NBUF)

            dma_wait(j, p)
            r0 = lo8 + i * R
            a = jnp.maximum(lo - r0, 0)
            b2 = jnp.minimum(hi - r0, R)

            def row_body(k, vecs2):
                return tuple(vecs2[cc] + buf[p, k, pl.ds(cc * L, L)]
                             for cc in range(CB))

            return lax.fori_loop(a, b2, row_body, vecs)

        vecs = lax.fori_loop(0, nch, chunk_body,
                             tuple(zeros16 for _ in range(CB)))
        for cc in range(CB):
            acc[j, pl.ds(cc * L, L)] = vecs[cc]
        return carry

    lax.fori_loop(0, B, batch_body, 0)

    pltpu.sync_copy(acc, shared.at[s])
    plsc.subcore_barrier()

    pltpu.sync_copy(shared.at[0, s], obuf)

    def red_body(w, carry):
        pltpu.sync_copy(shared.at[w, s], tmp)

        def add_body(i, carry2):
            obuf[pl.ds(i * L, L)] = obuf[pl.ds(i * L, L)] + tmp[pl.ds(i * L, L)]
            return carry2

        lax.fori_loop(0, CB, add_body, 0)
        return carry

    lax.fori_loop(1, NS, red_body, 0)

    invs = plsc.load_gather(invv, [jnp.full((L,), s, jnp.int32)])

    def mul_body(i, carry):
        obuf[pl.ds(i * L, L)] = obuf[pl.ds(i * L, L)] * invs
        return carry

    lax.fori_loop(0, CB, mul_body, 0)
    pltpu.sync_copy(obuf, out_h.at[s, 0, pl.ds(ch0, HALF)])


_mean_sc = functools.partial(
    pl.kernel,
    out_type=jax.ShapeDtypeStruct((B, 1, C), jnp.float32),
    mesh=plsc.VectorSubcoreMesh(core_axis_name="c", subcore_axis_name="s"),
    compiler_params=pltpu.CompilerParams(needs_layout_passes=False),
    scratch_types=[
        pltpu.VMEM((NBUF, R, HALF), jnp.float32),
        pltpu.VMEM((B, HALF), jnp.float32),
        pltpu.VMEM((L,), jnp.int32),
        pltpu.VMEM((L,), jnp.float32),
        pltpu.VMEM((HALF,), jnp.float32),
        pltpu.VMEM((HALF,), jnp.float32),
        pltpu.VMEM_SHARED((NS, B, HALF), jnp.float32),
        pltpu.SemaphoreType.DMA((NBUF,)),
    ],
)(_sc_body)


def kernel(value, sent_len):
    return _mean_sc(value, sent_len)
